# baseline (speedup 1.0000x reference)
import functools

import jax
import jax.numpy as jnp
from jax import lax
from jax.experimental import pallas as pl
from jax.experimental.pallas import tpu as pltpu
from jax.experimental.pallas import tpu_sc as plsc

T = 12
IN = 128
H = 128
NB = 256

N_LSOA, N_MED, N_COARSE = 10000, 769, 192
NP_LSOA, NP_MED, NP_COARSE = 10240, 1024, 256
N_PAD = NP_LSOA + NP_MED + NP_COARSE
OFF_MED, OFF_COARSE = NP_LSOA, NP_LSOA + NP_MED
E_ALL = 320000 + 4096 + 1024
EPB = 128
IBC = 16
NTILES = 16
NCORES = 2
EB_DEG = 80
EB_AGG = NCORES * EB_DEG
E_PAD = NCORES * NTILES * EB_DEG * EPB
ROWS_T = N_PAD // NTILES
PAD_NODE = N_PAD - 1
TPC = T // NCORES

_MESH = plsc.VectorSubcoreMesh(core_axis_name="c", subcore_axis_name="s")


def _deg_body(dst_hbm, ones_hbm, zeros_hbm, out_hbm, dst_v, ones_v, acc_sh):
    c = lax.axis_index("c")
    s = lax.axis_index("s")
    pltpu.sync_copy(ones_hbm, ones_v)
    pltpu.sync_copy(zeros_hbm, acc_sh.at[pl.ds(s * ROWS_T, ROWS_T)])
    plsc.subcore_barrier()

    def outer(sb, carry):
        pltpu.sync_copy(dst_hbm.at[c].at[s].at[pl.ds(sb * IBC, IBC)], dst_v)

        def body(b, carry2):
            pltpu.sync_copy(ones_v, acc_sh.at[dst_v.at[b]], add=True)
            return carry2

        lax.fori_loop(0, IBC, body, 0)
        return carry

    lax.fori_loop(0, EB_DEG // IBC, outer, 0)
    plsc.subcore_barrier()
    pltpu.sync_copy(acc_sh.at[pl.ds(s * ROWS_T, ROWS_T)],
                    out_hbm.at[c].at[pl.ds(s * ROWS_T, ROWS_T)])


_deg = functools.partial(
    pl.kernel,
    out_type=jax.ShapeDtypeStruct((NCORES, N_PAD, IN), jnp.float32),
    mesh=_MESH,
    scratch_types=[
        pltpu.VMEM((IBC, EPB), jnp.int32),
        pltpu.VMEM((EPB, IN), jnp.float32),
        pltpu.VMEM_SHARED((N_PAD, IN), jnp.float32),
    ],
)(_deg_body)


PAIRS = IBC // 2


def _agg_body(xs_hbm, src_hbm, dst_hbm, out_hbm, src_v, dst_v, rows0, rows1,
              acc_sh, g0, g1, s0, s1):
    c = lax.axis_index("c")
    s = lax.axis_index("s")
    slab = pl.ds(s * ROWS_T, ROWS_T)
    for t in range(T):
        @pl.when(c == (t // TPC))
        def _chunk(t=t):
            xst = xs_hbm.at[t]
            pltpu.sync_copy(xst.at[slab], acc_sh.at[slab])
            plsc.subcore_barrier()

            def outer(sb, carry):
                pltpu.sync_copy(src_hbm.at[s].at[pl.ds(sb * IBC, IBC)], src_v)
                pltpu.sync_copy(dst_hbm.at[s].at[pl.ds(sb * IBC, IBC)], dst_v)
                pltpu.async_copy(xst.at[src_v.at[0]], rows0, g0)

                def pair(i2, carry2):
                    b0 = 2 * i2
                    pltpu.make_async_copy(xst.at[src_v.at[0]], rows0,
                                          g0).wait()
                    pltpu.async_copy(rows0, acc_sh.at[dst_v.at[b0]], s0,
                                     add=True)

                    @pl.when(i2 > 0)
                    def _():
                        pltpu.make_async_copy(
                            rows1, acc_sh.at[dst_v.at[0]], s1).wait()

                    pltpu.async_copy(xst.at[src_v.at[b0 + 1]], rows1, g1)
                    pltpu.make_async_copy(xst.at[src_v.at[0]], rows1,
                                          g1).wait()
                    pltpu.async_copy(rows1, acc_sh.at[dst_v.at[b0 + 1]], s1,
                                     add=True)
                    pltpu.make_async_copy(rows0, acc_sh.at[dst_v.at[0]],
                                          s0).wait()

                    @pl.when(i2 < PAIRS - 1)
                    def _():
                        pltpu.async_copy(xst.at[src_v.at[b0 + 2]], rows0, g0)

                    return carry2

                lax.fori_loop(0, PAIRS, pair, 0)
                pltpu.make_async_copy(rows1, acc_sh.at[dst_v.at[0]],
                                      s1).wait()
                return carry

            lax.fori_loop(0, EB_AGG // IBC, outer, 0)
            plsc.subcore_barrier()
            pltpu.sync_copy(acc_sh.at[slab], out_hbm.at[t].at[slab])
            plsc.subcore_barrier()


_agg = functools.partial(
    pl.kernel,
    out_type=jax.ShapeDtypeStruct((T, N_PAD, IN), jnp.float32),
    mesh=_MESH,
    scratch_types=[
        pltpu.VMEM((IBC, EPB), jnp.int32),
        pltpu.VMEM((IBC, EPB), jnp.int32),
        pltpu.VMEM((EPB, IN), jnp.float32),
        pltpu.VMEM((EPB, IN), jnp.float32),
        pltpu.VMEM_SHARED((N_PAD, IN), jnp.float32),
        pltpu.SemaphoreType.DMA,
        pltpu.SemaphoreType.DMA,
        pltpu.SemaphoreType.DMA,
        pltpu.SemaphoreType.DMA,
    ],
)(_agg_body)


NBA = 256


def _scale_body(x_ref, degp_ref, xs_ref, dinv_ref):
    deg = jnp.sum(degp_ref[...], axis=(0, 2)) * (1.0 / IN) + 1.0
    dinv = lax.rsqrt(deg)[:, None]
    xs_ref[...] = jnp.swapaxes(x_ref[...] * dinv[:, None, :], 0, 1)
    dinv_ref[...] = jnp.broadcast_to(dinv, (NBA, IN))


_scale = pl.pallas_call(
    _scale_body,
    grid=(N_PAD // NBA,),
    in_specs=[
        pl.BlockSpec((NBA, T, IN), lambda i: (i, 0, 0)),
        pl.BlockSpec((2, NBA, IN), lambda i: (0, i, 0)),
    ],
    out_specs=[
        pl.BlockSpec((T, NBA, IN), lambda i: (0, i, 0)),
        pl.BlockSpec((NBA, IN), lambda i: (i, 0)),
    ],
    out_shape=[
        jax.ShapeDtypeStruct((T, N_PAD, IN), jnp.float32),
        jax.ShapeDtypeStruct((N_PAD, IN), jnp.float32),
    ],
    compiler_params=pltpu.CompilerParams(
        dimension_semantics=("parallel",)),
)


def _head_body(agg_ref, dinv_ref, gw_ref, gb_ref, wih_ref, whh_ref, bias_ref,
               fw_ref, fb_ref, out_ref, gx_scr):
    dinv = dinv_ref[...]
    for t in range(T):
        a = agg_ref[t] * dinv
        h = jnp.maximum(
            jnp.dot(a, gw_ref[...], preferred_element_type=jnp.float32)
            + gb_ref[...], 0.0)
        gx_scr[t] = (
            jnp.dot(h, wih_ref[...], preferred_element_type=jnp.float32)
            + bias_ref[...])
    h = jnp.zeros((NB, H), dtype=jnp.float32)
    c = jnp.zeros((NB, H), dtype=jnp.float32)
    for t in range(T):
        g = gx_scr[t] + jnp.dot(h, whh_ref[...],
                                preferred_element_type=jnp.float32)
        i = jax.nn.sigmoid(g[:, 0:H])
        f = jax.nn.sigmoid(g[:, H:2 * H])
        gg = jnp.tanh(g[:, 2 * H:3 * H])
        o = jax.nn.sigmoid(g[:, 3 * H:4 * H])
        c = f * c + i * gg
        h = o * jnp.tanh(c)
    pred = jnp.sum(h * fw_ref[...], axis=1, keepdims=True)
    out_ref[...] = pred + fb_ref[...]


def _head(agg, dinvb, gw, gb, wih_t, whh_t, bias, fw, fb128, off, n_pad):
    ob = off // NB
    return pl.pallas_call(
        _head_body,
        grid=(n_pad // NB,),
        in_specs=[
            pl.BlockSpec((T, NB, IN), lambda i: (0, ob + i, 0)),
            pl.BlockSpec((NB, IN), lambda i: (ob + i, 0)),
            pl.BlockSpec((IN, H), lambda i: (0, 0)),
            pl.BlockSpec((1, H), lambda i: (0, 0)),
            pl.BlockSpec((H, 4 * H), lambda i: (0, 0)),
            pl.BlockSpec((H, 4 * H), lambda i: (0, 0)),
            pl.BlockSpec((1, 4 * H), lambda i: (0, 0)),
            pl.BlockSpec((1, H), lambda i: (0, 0)),
            pl.BlockSpec((1, H), lambda i: (0, 0)),
        ],
        out_specs=pl.BlockSpec((NB, H), lambda i: (i, 0)),
        out_shape=jax.ShapeDtypeStruct((n_pad, H), jnp.float32),
        scratch_shapes=[pltpu.VMEM((T, NB, 4 * H), jnp.float32)],
        compiler_params=pltpu.CompilerParams(
            dimension_semantics=("parallel",)),
    )(agg, dinvb, gw, gb, wih_t, whh_t, bias, fw, fb128)


def _run_head(agg, dinvb, gw, gb, wih, whh, bih, bhh, fw, fb, off, n_pad, n):
    out = _head(agg, dinvb, gw, gb.reshape(1, H), wih.T, whh.T,
                (bih + bhh).reshape(1, 4 * H), fw,
                jnp.broadcast_to(fb.reshape(1, 1), (1, H)), off, n_pad)
    return out[:n, 0]


def kernel(lsoa_x, lsoa_edge_index, med_x, med_edge_index, coarse_x,
           coarse_edge_index, lsoa_gcn_W, lsoa_gcn_b, lsoa_lstm_Wih,
           lsoa_lstm_Whh, lsoa_lstm_bih, lsoa_lstm_bhh, lsoa_fc_W, lsoa_fc_b,
           med_gcn_W, med_gcn_b, med_lstm_Wih, med_lstm_Whh, med_lstm_bih,
           med_lstm_bhh, med_fc_W, med_fc_b, coarse_gcn_W, coarse_gcn_b,
           coarse_lstm_Wih, coarse_lstm_Whh, coarse_lstm_bih, coarse_lstm_bhh,
           coarse_fc_W, coarse_fc_b):
    x_all = jnp.zeros((N_PAD, T, IN), jnp.float32)
    x_all = lax.dynamic_update_slice(x_all, lsoa_x, (0, 0, 0))
    x_all = lax.dynamic_update_slice(x_all, med_x, (OFF_MED, 0, 0))
    x_all = lax.dynamic_update_slice(x_all, coarse_x, (OFF_COARSE, 0, 0))
    pad_e = jnp.full((E_PAD - E_ALL,), PAD_NODE, jnp.int32)
    src_all = jnp.concatenate([
        lsoa_edge_index[0], med_edge_index[0] + OFF_MED,
        coarse_edge_index[0] + OFF_COARSE, pad_e])
    dst_all = jnp.concatenate([
        lsoa_edge_index[1], med_edge_index[1] + OFF_MED,
        coarse_edge_index[1] + OFF_COARSE, pad_e])

    degp = _deg(dst_all.reshape(NCORES, NTILES, EB_DEG, EPB),
                jnp.ones((EPB, IN), jnp.float32),
                jnp.zeros((ROWS_T, IN), jnp.float32))
    xs, dinvb = _scale(x_all, degp)
    agg = _agg(xs, src_all.reshape(NTILES, EB_AGG, EPB),
               dst_all.reshape(NTILES, EB_AGG, EPB))

    pred_lsoa = _run_head(agg, dinvb, lsoa_gcn_W, lsoa_gcn_b, lsoa_lstm_Wih,
                          lsoa_lstm_Whh, lsoa_lstm_bih, lsoa_lstm_bhh,
                          lsoa_fc_W, lsoa_fc_b, 0, NP_LSOA, N_LSOA)
    pred_med = _run_head(agg, dinvb, med_gcn_W, med_gcn_b, med_lstm_Wih,
                         med_lstm_Whh, med_lstm_bih, med_lstm_bhh, med_fc_W,
                         med_fc_b, OFF_MED, NP_MED, N_MED)
    pred_coarse = _run_head(agg, dinvb, coarse_gcn_W, coarse_gcn_b,
                            coarse_lstm_Wih, coarse_lstm_Whh, coarse_lstm_bih,
                            coarse_lstm_bhh, coarse_fc_W, coarse_fc_b,
                            OFF_COARSE, NP_COARSE, N_COARSE)
    return (pred_lsoa, pred_med, pred_coarse)


# speedup vs baseline: 1.1015x; 1.1015x over previous
import functools

import jax
import jax.numpy as jnp
from jax import lax
from jax.experimental import pallas as pl
from jax.experimental.pallas import tpu as pltpu
from jax.experimental.pallas import tpu_sc as plsc

T = 12
IN = 128
H = 128
NB = 256

N_LSOA, N_MED, N_COARSE = 10000, 769, 192
NP_LSOA, NP_MED, NP_COARSE = 10240, 1024, 256
N_PAD = NP_LSOA + NP_MED + NP_COARSE
OFF_MED, OFF_COARSE = NP_LSOA, NP_LSOA + NP_MED
E_ALL = 320000 + 4096 + 1024
EPB = 128
IBC = 16
NTILES = 16
NCORES = 2
EB_DEG = 80
EB_AGG = NCORES * EB_DEG
E_PAD = NCORES * NTILES * EB_DEG * EPB
ROWS_T = N_PAD // NTILES
PAD_NODE = N_PAD - 1
TPC = T // NCORES
DW = 32

_MESH = plsc.VectorSubcoreMesh(core_axis_name="c", subcore_axis_name="s")


def _deg_body(dst_hbm, ones_hbm, zeros_hbm, out_hbm, dst_v, ones_v, acc_sh):
    c = lax.axis_index("c")
    s = lax.axis_index("s")
    pltpu.sync_copy(ones_hbm, ones_v)
    pltpu.sync_copy(zeros_hbm, acc_sh.at[pl.ds(s * ROWS_T, ROWS_T)])
    plsc.subcore_barrier()

    def outer(sb, carry):
        pltpu.sync_copy(dst_hbm.at[c].at[s].at[pl.ds(sb * IBC, IBC)], dst_v)

        def body(b, carry2):
            pltpu.sync_copy(ones_v, acc_sh.at[dst_v.at[b]], add=True)
            return carry2

        lax.fori_loop(0, IBC, body, 0)
        return carry

    lax.fori_loop(0, EB_DEG // IBC, outer, 0)
    plsc.subcore_barrier()
    pltpu.sync_copy(acc_sh.at[pl.ds(s * ROWS_T, ROWS_T)],
                    out_hbm.at[c].at[pl.ds(s * ROWS_T, ROWS_T)])


_deg = functools.partial(
    pl.kernel,
    out_type=jax.ShapeDtypeStruct((NCORES, N_PAD, DW), jnp.float32),
    mesh=_MESH,
    scratch_types=[
        pltpu.VMEM((IBC, EPB), jnp.int32),
        pltpu.VMEM((EPB, DW), jnp.float32),
        pltpu.VMEM_SHARED((N_PAD, DW), jnp.float32),
    ],
)(_deg_body)


PAIRS = IBC // 2


def _agg_body(xs_hbm, src_hbm, dst_hbm, out_hbm, src_v, dst_v, rows0, rows1,
              acc_sh, sem0, sem1):
    c = lax.axis_index("c")
    s = lax.axis_index("s")
    slab = pl.ds(s * ROWS_T, ROWS_T)
    for t in range(T):
        @pl.when(c == (t // TPC))
        def _chunk(t=t):
            xst = xs_hbm.at[t]
            pltpu.sync_copy(xst.at[slab], acc_sh.at[slab])
            plsc.subcore_barrier()

            def outer(sb, carry):
                pltpu.sync_copy(src_hbm.at[s].at[pl.ds(sb * IBC, IBC)], src_v)
                pltpu.sync_copy(dst_hbm.at[s].at[pl.ds(sb * IBC, IBC)], dst_v)
                pltpu.async_copy(xst.at[src_v.at[0]], rows0, sem0)

                def pair(i2, carry2):
                    b0 = 2 * i2
                    pltpu.async_copy(xst.at[src_v.at[b0 + 1]], rows1, sem1)
                    pltpu.make_async_copy(xst.at[src_v.at[0]], rows0,
                                          sem0).wait()
                    pltpu.sync_copy(rows0, acc_sh.at[dst_v.at[b0]], add=True)

                    @pl.when(i2 < PAIRS - 1)
                    def _():
                        pltpu.async_copy(xst.at[src_v.at[b0 + 2]], rows0,
                                         sem0)

                    pltpu.make_async_copy(xst.at[src_v.at[0]], rows1,
                                          sem1).wait()
                    pltpu.sync_copy(rows1, acc_sh.at[dst_v.at[b0 + 1]],
                                    add=True)
                    return carry2

                lax.fori_loop(0, PAIRS, pair, 0)
                return carry

            lax.fori_loop(0, EB_AGG // IBC, outer, 0)
            plsc.subcore_barrier()
            pltpu.sync_copy(acc_sh.at[slab], out_hbm.at[t].at[slab])
            plsc.subcore_barrier()


_agg = functools.partial(
    pl.kernel,
    out_type=jax.ShapeDtypeStruct((T, N_PAD, IN), jnp.float32),
    mesh=_MESH,
    scratch_types=[
        pltpu.VMEM((IBC, EPB), jnp.int32),
        pltpu.VMEM((IBC, EPB), jnp.int32),
        pltpu.VMEM((EPB, IN), jnp.float32),
        pltpu.VMEM((EPB, IN), jnp.float32),
        pltpu.VMEM_SHARED((N_PAD, IN), jnp.float32),
        pltpu.SemaphoreType.DMA,
        pltpu.SemaphoreType.DMA,
    ],
)(_agg_body)


NBA = 256


def _scale_body(x_ref, degp_ref, xs_ref, dinv_ref):
    deg = jnp.sum(degp_ref[...], axis=(0, 2)) * (1.0 / DW) + 1.0
    dinv = lax.rsqrt(deg)[:, None]
    xs_ref[...] = jnp.swapaxes(x_ref[...] * dinv[:, None, :], 0, 1)
    dinv_ref[...] = jnp.broadcast_to(dinv, (NBA, IN))


_scale = pl.pallas_call(
    _scale_body,
    grid=(N_PAD // NBA,),
    in_specs=[
        pl.BlockSpec((NBA, T, IN), lambda i: (i, 0, 0)),
        pl.BlockSpec((2, NBA, DW), lambda i: (0, i, 0)),
    ],
    out_specs=[
        pl.BlockSpec((T, NBA, IN), lambda i: (0, i, 0)),
        pl.BlockSpec((NBA, IN), lambda i: (i, 0)),
    ],
    out_shape=[
        jax.ShapeDtypeStruct((T, N_PAD, IN), jnp.float32),
        jax.ShapeDtypeStruct((N_PAD, IN), jnp.float32),
    ],
    compiler_params=pltpu.CompilerParams(
        dimension_semantics=("parallel",)),
)


def _head_body(agg_ref, dinv_ref, gw_ref, gb_ref, wih_ref, whh_ref, bias_ref,
               fw_ref, fb_ref, out_ref, gx_scr):
    dinv = dinv_ref[...]
    gw = gw_ref[...].astype(jnp.bfloat16)
    wih = wih_ref[...].astype(jnp.bfloat16)
    whh = whh_ref[...].astype(jnp.bfloat16)
    for t in range(T):
        a = (agg_ref[t] * dinv).astype(jnp.bfloat16)
        h = jnp.maximum(
            jnp.dot(a, gw, preferred_element_type=jnp.float32)
            + gb_ref[...], 0.0)
        gx_scr[t] = (
            jnp.dot(h.astype(jnp.bfloat16), wih,
                    preferred_element_type=jnp.float32)
            + bias_ref[...])
    h = jnp.zeros((NB, H), dtype=jnp.float32)
    c = jnp.zeros((NB, H), dtype=jnp.float32)
    for t in range(T):
        g = gx_scr[t] + jnp.dot(h.astype(jnp.bfloat16), whh,
                                preferred_element_type=jnp.float32)
        i = jax.nn.sigmoid(g[:, 0:H])
        f = jax.nn.sigmoid(g[:, H:2 * H])
        gg = jnp.tanh(g[:, 2 * H:3 * H])
        o = jax.nn.sigmoid(g[:, 3 * H:4 * H])
        c = f * c + i * gg
        h = o * jnp.tanh(c)
    pred = jnp.sum(h * fw_ref[...], axis=1, keepdims=True)
    out_ref[...] = pred + fb_ref[...]


def _head(agg, dinvb, gw, gb, wih_t, whh_t, bias, fw, fb128, off, n_pad):
    ob = off // NB
    return pl.pallas_call(
        _head_body,
        grid=(n_pad // NB,),
        in_specs=[
            pl.BlockSpec((T, NB, IN), lambda i: (0, ob + i, 0)),
            pl.BlockSpec((NB, IN), lambda i: (ob + i, 0)),
            pl.BlockSpec((IN, H), lambda i: (0, 0)),
            pl.BlockSpec((1, H), lambda i: (0, 0)),
            pl.BlockSpec((H, 4 * H), lambda i: (0, 0)),
            pl.BlockSpec((H, 4 * H), lambda i: (0, 0)),
            pl.BlockSpec((1, 4 * H), lambda i: (0, 0)),
            pl.BlockSpec((1, H), lambda i: (0, 0)),
            pl.BlockSpec((1, H), lambda i: (0, 0)),
        ],
        out_specs=pl.BlockSpec((NB, H), lambda i: (i, 0)),
        out_shape=jax.ShapeDtypeStruct((n_pad, H), jnp.float32),
        scratch_shapes=[pltpu.VMEM((T, NB, 4 * H), jnp.float32)],
        compiler_params=pltpu.CompilerParams(
            dimension_semantics=("parallel",)),
    )(agg, dinvb, gw, gb, wih_t, whh_t, bias, fw, fb128)


def _run_head(agg, dinvb, gw, gb, wih, whh, bih, bhh, fw, fb, off, n_pad, n):
    out = _head(agg, dinvb, gw, gb.reshape(1, H), wih.T, whh.T,
                (bih + bhh).reshape(1, 4 * H), fw,
                jnp.broadcast_to(fb.reshape(1, 1), (1, H)), off, n_pad)
    return out[:n, 0]


def kernel(lsoa_x, lsoa_edge_index, med_x, med_edge_index, coarse_x,
           coarse_edge_index, lsoa_gcn_W, lsoa_gcn_b, lsoa_lstm_Wih,
           lsoa_lstm_Whh, lsoa_lstm_bih, lsoa_lstm_bhh, lsoa_fc_W, lsoa_fc_b,
           med_gcn_W, med_gcn_b, med_lstm_Wih, med_lstm_Whh, med_lstm_bih,
           med_lstm_bhh, med_fc_W, med_fc_b, coarse_gcn_W, coarse_gcn_b,
           coarse_lstm_Wih, coarse_lstm_Whh, coarse_lstm_bih, coarse_lstm_bhh,
           coarse_fc_W, coarse_fc_b):
    x_all = jnp.zeros((N_PAD, T, IN), jnp.float32)
    x_all = lax.dynamic_update_slice(x_all, lsoa_x, (0, 0, 0))
    x_all = lax.dynamic_update_slice(x_all, med_x, (OFF_MED, 0, 0))
    x_all = lax.dynamic_update_slice(x_all, coarse_x, (OFF_COARSE, 0, 0))
    pad_e = jnp.full((E_PAD - E_ALL,), PAD_NODE, jnp.int32)
    src_all = jnp.concatenate([
        lsoa_edge_index[0], med_edge_index[0] + OFF_MED,
        coarse_edge_index[0] + OFF_COARSE, pad_e])
    dst_all = jnp.concatenate([
        lsoa_edge_index[1], med_edge_index[1] + OFF_MED,
        coarse_edge_index[1] + OFF_COARSE, pad_e])

    degp = _deg(dst_all.reshape(NCORES, NTILES, EB_DEG, EPB),
                jnp.ones((EPB, DW), jnp.float32),
                jnp.zeros((ROWS_T, DW), jnp.float32))
    xs, dinvb = _scale(x_all, degp)
    agg = _agg(xs, src_all.reshape(NTILES, EB_AGG, EPB),
               dst_all.reshape(NTILES, EB_AGG, EPB))

    pred_lsoa = _run_head(agg, dinvb, lsoa_gcn_W, lsoa_gcn_b, lsoa_lstm_Wih,
                          lsoa_lstm_Whh, lsoa_lstm_bih, lsoa_lstm_bhh,
                          lsoa_fc_W, lsoa_fc_b, 0, NP_LSOA, N_LSOA)
    pred_med = _run_head(agg, dinvb, med_gcn_W, med_gcn_b, med_lstm_Wih,
                         med_lstm_Whh, med_lstm_bih, med_lstm_bhh, med_fc_W,
                         med_fc_b, OFF_MED, NP_MED, N_MED)
    pred_coarse = _run_head(agg, dinvb, coarse_gcn_W, coarse_gcn_b,
                            coarse_lstm_Wih, coarse_lstm_Whh, coarse_lstm_bih,
                            coarse_lstm_bhh, coarse_fc_W, coarse_fc_b,
                            OFF_COARSE, NP_COARSE, N_COARSE)
    return (pred_lsoa, pred_med, pred_coarse)
